# baseline (speedup 1.0000x reference)
import functools

import jax
import jax.numpy as jnp
from jax import lax
from jax.experimental import pallas as pl
from jax.experimental.pallas import tpu as pltpu
from jax.experimental.pallas import tpu_sc as plsc

NUM_EXPERTS = 64
TOP_K = 8
HIDDEN = 768
TOKENS = 32768

BLOCK_T = 4096
NUM_WORKERS = 32
TOK_PER_W = TOKENS // NUM_WORKERS
CHUNK = 128
N_CHUNKS = TOK_PER_W // CHUNK
L = 16
NV = NUM_EXPERTS // L


HALF_H = HIDDEN // 2


def _logits_block(x1_ref, x2_ref, w1_ref, w2_ref, o_ref):
    dn = (((1,), (1,)), ((), ()))
    logits = (
        lax.dot_general(x1_ref[...], w1_ref[...], dn,
                        preferred_element_type=jnp.float32)
        + lax.dot_general(x2_ref[...], w2_ref[...], dn,
                          preferred_element_type=jnp.float32))
    row_max = jnp.max(logits, axis=-1, keepdims=True)
    ex = jnp.exp(logits - row_max)
    o_ref[...] = ex / jnp.sum(ex, axis=-1, keepdims=True)


def _tc_logits(x, W):
    return pl.pallas_call(
        _logits_block,
        grid=(TOKENS // BLOCK_T,),
        in_specs=[
            pl.BlockSpec((BLOCK_T, HALF_H), lambda i: (i, 0)),
            pl.BlockSpec((BLOCK_T, HALF_H), lambda i: (i, 1)),
            pl.BlockSpec((NUM_EXPERTS, HALF_H), lambda i: (0, 0)),
            pl.BlockSpec((NUM_EXPERTS, HALF_H), lambda i: (0, 1)),
        ],
        out_specs=pl.BlockSpec((BLOCK_T, NUM_EXPERTS), lambda i: (i, 0)),
        out_shape=jax.ShapeDtypeStruct((TOKENS, NUM_EXPERTS), jnp.float32),
    )(x, x, W, W)


def _srt(v, iota):
    k, _ = plsc.sort_key_val(v, iota, descending=True)
    return k


def _tok_compute(buf_l, buf_g, buf_m, tk, iota):
    pr = [buf_l[tk, pl.ds(j * L, L)] for j in range(NV)]
    s0, s1, s2, s3 = (_srt(prj, iota) for prj in pr)
    m01 = _srt(jnp.maximum(s0, lax.rev(s1, (0,))), iota)
    m23 = _srt(jnp.maximum(s2, lax.rev(s3, (0,))), iota)
    mm = _srt(jnp.maximum(m01, lax.rev(m23, (0,))), iota)
    t = jnp.max(jnp.where(iota == TOP_K - 1, mm, -1.0))
    cnt_gt = plsc.all_reduce_population_count(mm > t)
    tie_seen = jnp.zeros((L,), jnp.int32)
    for j in range(NV):
        eqj = pr[j] == t
        rj = lax.cumsum(eqj.astype(jnp.int32)) + tie_seen
        selj = (pr[j] > t) | (eqj & ((rj + cnt_gt) <= TOP_K))
        tie_seen = tie_seen + plsc.all_reduce_population_count(eqj)
        buf_g[tk, pl.ds(j * L, L)] = jnp.where(selj, pr[j], 0.0)
        buf_m[tk, pl.ds(j * L, L)] = selj.astype(jnp.int32)


UNROLL = 4


def _route_body(logits_hbm, gates_hbm, map_hbm,
                bl0, bl1, bg0, bg1, bm0, bm1,
                si0, si1, sg0, sg1, sm0, sm1):
    cid = lax.axis_index("c")
    sid = lax.axis_index("s")
    wid = sid * 2 + cid
    base = wid * TOK_PER_W
    iota = lax.iota(jnp.int32, L)
    bl, bg, bm = [bl0, bl1], [bg0, bg1], [bm0, bm1]
    si, sg, sm = [si0, si1], [sg0, sg1], [sm0, sm1]

    def start_in(ci):
        return pltpu.async_copy(
            logits_hbm.at[pl.ds(base + ci * CHUNK, CHUNK)], bl[ci % 2],
            si[ci % 2])

    in_cp = {0: start_in(0)}
    out_cp = {}
    for ci in range(N_CHUNKS):
        s = ci % 2
        if ci + 1 < N_CHUNKS:
            in_cp[ci + 1] = start_in(ci + 1)
        in_cp[ci].wait()
        if ci >= 2:
            gcp, mcp = out_cp[ci - 2]
            gcp.wait()
            mcp.wait()

        def tok_body(ti, carry):
            for u in range(UNROLL):
                _tok_compute(bl[s], bg[s], bm[s], ti * UNROLL + u, iota)
            return carry

        lax.fori_loop(0, CHUNK // UNROLL, tok_body, 0)
        cbase = base + ci * CHUNK
        out_cp[ci] = (
            pltpu.async_copy(bg[s], gates_hbm.at[pl.ds(cbase, CHUNK)], sg[s]),
            pltpu.async_copy(bm[s], map_hbm.at[pl.ds(cbase, CHUNK)], sm[s]),
        )
    for ci in (N_CHUNKS - 2, N_CHUNKS - 1):
        gcp, mcp = out_cp[ci]
        gcp.wait()
        mcp.wait()


def _sc_route(logits):
    mesh = plsc.VectorSubcoreMesh(core_axis_name="c", subcore_axis_name="s")
    fn = pl.kernel(
        _route_body,
        out_type=[
            jax.ShapeDtypeStruct((TOKENS, NUM_EXPERTS), jnp.float32),
            jax.ShapeDtypeStruct((TOKENS, NUM_EXPERTS), jnp.int32),
        ],
        mesh=mesh,
        compiler_params=pltpu.CompilerParams(needs_layout_passes=False),
        scratch_types=[
            pltpu.VMEM((CHUNK, NUM_EXPERTS), jnp.float32),
            pltpu.VMEM((CHUNK, NUM_EXPERTS), jnp.float32),
            pltpu.VMEM((CHUNK, NUM_EXPERTS), jnp.float32),
            pltpu.VMEM((CHUNK, NUM_EXPERTS), jnp.float32),
            pltpu.VMEM((CHUNK, NUM_EXPERTS), jnp.int32),
            pltpu.VMEM((CHUNK, NUM_EXPERTS), jnp.int32),
            pltpu.SemaphoreType.DMA,
            pltpu.SemaphoreType.DMA,
            pltpu.SemaphoreType.DMA,
            pltpu.SemaphoreType.DMA,
            pltpu.SemaphoreType.DMA,
            pltpu.SemaphoreType.DMA,
        ],
    )
    return fn(logits)


@jax.jit
def kernel(x, W):
    logits = _tc_logits(x, W)
    gates, topk_map = _sc_route(logits)
    return (gates, topk_map)


# speedup vs baseline: 1.1989x; 1.1989x over previous
import functools

import jax
import jax.numpy as jnp
from jax import lax
from jax.experimental import pallas as pl
from jax.experimental.pallas import tpu as pltpu
from jax.experimental.pallas import tpu_sc as plsc

NUM_EXPERTS = 64
TOP_K = 8
HIDDEN = 768
TOKENS = 32768

BLOCK_T = 4096
NUM_WORKERS = 32
TOK_PER_W = TOKENS // NUM_WORKERS
CHUNK = 128
N_CHUNKS = TOK_PER_W // CHUNK
L = 16
NV = NUM_EXPERTS // L


HALF_H = HIDDEN // 2


def _logits_block(x1_ref, x2_ref, w1_ref, w2_ref, o_ref):
    dn = (((1,), (1,)), ((), ()))
    logits = (
        lax.dot_general(x1_ref[...], w1_ref[...], dn,
                        preferred_element_type=jnp.float32)
        + lax.dot_general(x2_ref[...], w2_ref[...], dn,
                          preferred_element_type=jnp.float32))
    row_max = jnp.max(logits, axis=-1, keepdims=True)
    ex = jnp.exp(logits - row_max)
    o_ref[...] = ex / jnp.sum(ex, axis=-1, keepdims=True)


def _tc_logits(x, W):
    return pl.pallas_call(
        _logits_block,
        grid=(TOKENS // BLOCK_T,),
        in_specs=[
            pl.BlockSpec((BLOCK_T, HALF_H), lambda i: (i, 0)),
            pl.BlockSpec((BLOCK_T, HALF_H), lambda i: (i, 1)),
            pl.BlockSpec((NUM_EXPERTS, HALF_H), lambda i: (0, 0)),
            pl.BlockSpec((NUM_EXPERTS, HALF_H), lambda i: (0, 1)),
        ],
        out_specs=pl.BlockSpec((BLOCK_T, NUM_EXPERTS), lambda i: (i, 0)),
        out_shape=jax.ShapeDtypeStruct((TOKENS, NUM_EXPERTS), jnp.float32),
    )(x, x, W, W)


def _srt(v, iota):
    k, _ = plsc.sort_key_val(v, iota, descending=True)
    return k


def _tok_compute(buf_l, buf_g, buf_m, tk, iota):
    pr = [buf_l[tk, pl.ds(j * L, L)] for j in range(NV)]
    s0, s1, s2, s3 = (_srt(prj, iota) for prj in pr)
    m01 = _srt(jnp.maximum(s0, lax.rev(s1, (0,))), iota)
    m23 = _srt(jnp.maximum(s2, lax.rev(s3, (0,))), iota)
    mm = _srt(jnp.maximum(m01, lax.rev(m23, (0,))), iota)
    t = jnp.max(jnp.where(iota == TOP_K - 1, mm, -1.0))
    cnt_gt = plsc.all_reduce_population_count(mm > t)
    tie_seen = jnp.zeros((L,), jnp.int32)
    for j in range(NV):
        eqj = pr[j] == t
        rj = lax.cumsum(eqj.astype(jnp.int32)) + tie_seen
        selj = (pr[j] > t) | (eqj & ((rj + cnt_gt) <= TOP_K))
        tie_seen = tie_seen + plsc.all_reduce_population_count(eqj)
        buf_g[tk, pl.ds(j * L, L)] = jnp.where(selj, pr[j], 0.0)
        buf_m[tk, pl.ds(j * L, L)] = selj.astype(jnp.int32)


UNROLL = 2


def _route_body(logits_hbm, gates_hbm, map_hbm,
                bl0, bl1, bg0, bg1, bm0, bm1,
                si0, si1, sg0, sg1, sm0, sm1):
    cid = lax.axis_index("c")
    sid = lax.axis_index("s")
    wid = sid * 2 + cid
    base = wid * TOK_PER_W
    iota = lax.iota(jnp.int32, L)
    bl, bg, bm = [bl0, bl1], [bg0, bg1], [bm0, bm1]
    si, sg, sm = [si0, si1], [sg0, sg1], [sm0, sm1]

    def start_in(ci):
        return pltpu.async_copy(
            logits_hbm.at[pl.ds(base + ci * CHUNK, CHUNK)], bl[ci % 2],
            si[ci % 2])

    in_cp = {0: start_in(0)}
    out_cp = {}
    for ci in range(N_CHUNKS):
        s = ci % 2
        if ci + 1 < N_CHUNKS:
            in_cp[ci + 1] = start_in(ci + 1)
        in_cp[ci].wait()
        if ci >= 2:
            gcp, mcp = out_cp[ci - 2]
            gcp.wait()
            mcp.wait()

        @plsc.parallel_loop(0, CHUNK, 1, unroll=UNROLL)
        def tok_body(tk):
            _tok_compute(bl[s], bg[s], bm[s], tk, iota)
        cbase = base + ci * CHUNK
        out_cp[ci] = (
            pltpu.async_copy(bg[s], gates_hbm.at[pl.ds(cbase, CHUNK)], sg[s]),
            pltpu.async_copy(bm[s], map_hbm.at[pl.ds(cbase, CHUNK)], sm[s]),
        )
    for ci in (N_CHUNKS - 2, N_CHUNKS - 1):
        gcp, mcp = out_cp[ci]
        gcp.wait()
        mcp.wait()


def _sc_route(logits):
    mesh = plsc.VectorSubcoreMesh(core_axis_name="c", subcore_axis_name="s")
    fn = pl.kernel(
        _route_body,
        out_type=[
            jax.ShapeDtypeStruct((TOKENS, NUM_EXPERTS), jnp.float32),
            jax.ShapeDtypeStruct((TOKENS, NUM_EXPERTS), jnp.int32),
        ],
        mesh=mesh,
        compiler_params=pltpu.CompilerParams(needs_layout_passes=False),
        scratch_types=[
            pltpu.VMEM((CHUNK, NUM_EXPERTS), jnp.float32),
            pltpu.VMEM((CHUNK, NUM_EXPERTS), jnp.float32),
            pltpu.VMEM((CHUNK, NUM_EXPERTS), jnp.float32),
            pltpu.VMEM((CHUNK, NUM_EXPERTS), jnp.float32),
            pltpu.VMEM((CHUNK, NUM_EXPERTS), jnp.int32),
            pltpu.VMEM((CHUNK, NUM_EXPERTS), jnp.int32),
            pltpu.SemaphoreType.DMA,
            pltpu.SemaphoreType.DMA,
            pltpu.SemaphoreType.DMA,
            pltpu.SemaphoreType.DMA,
            pltpu.SemaphoreType.DMA,
            pltpu.SemaphoreType.DMA,
        ],
    )
    return fn(logits)


@jax.jit
def kernel(x, W):
    logits = _tc_logits(x, W)
    gates, topk_map = _sc_route(logits)
    return (gates, topk_map)
